# baseline (speedup 1.0000x reference)
import functools

import jax
import jax.numpy as jnp
from jax import lax
from jax.experimental import pallas as pl
from jax.experimental.pallas import tpu as pltpu
from jax.experimental.pallas import tpu_sc as plsc

BATCH = 1024
SEQ = 200
EMB = 64
NC = 2
NS = 16
NW = NC * NS

TOTAL = BATCH * SEQ
ROWS_PER_W = TOTAL // NW
IDX_MINOR = 100
IDX_ROWS_PER_CHUNK = SEQ // IDX_MINOR
CHUNK = SEQ
NCHUNK = ROWS_PER_W // CHUNK
IDX_ROWS_PER_W = ROWS_PER_W // IDX_MINOR

_mesh = plsc.VectorSubcoreMesh(
    core_axis_name="c", subcore_axis_name="s", num_cores=NC, num_subcores=NS
)


@functools.partial(
    pl.kernel,
    out_type=jax.ShapeDtypeStruct((TOTAL, EMB), jnp.float32),
    mesh=_mesh,
    compiler_params=pltpu.CompilerParams(use_tc_tiling_on_sc=False),
    scratch_types=[
        pltpu.VMEM((IDX_ROWS_PER_W, IDX_MINOR), jnp.int32),
        pltpu.VMEM((CHUNK, EMB), jnp.float32),
        pltpu.VMEM((CHUNK, EMB), jnp.float32),
        pltpu.VMEM((SEQ, EMB), jnp.float32),
        pltpu.SemaphoreType.DMA,
        pltpu.SemaphoreType.DMA,
        pltpu.SemaphoreType.DMA,
        pltpu.SemaphoreType.DMA,
    ],
)
def _embed_sc(idx_hbm, tok_hbm, pos_hbm, out_hbm,
              idx_v, rows_a, rows_b, pos_v, gsem_a, gsem_b, wsem_a, wsem_b):
    wid = lax.axis_index("s") * NC + lax.axis_index("c")
    idx_row0 = wid * IDX_ROWS_PER_W
    row0 = wid * ROWS_PER_W

    pltpu.sync_copy(idx_hbm.at[pl.ds(idx_row0, IDX_ROWS_PER_W)], idx_v)
    pltpu.sync_copy(pos_hbm, pos_v)

    rows = (rows_a, rows_b)
    gsem = (gsem_a, gsem_b)
    wsem = (wsem_a, wsem_b)

    def start_gather(c):
        p = c % 2
        gs = []
        for h in range(IDX_ROWS_PER_CHUNK):
            gs.append(pltpu.async_copy(
                tok_hbm.at[idx_v.at[c * IDX_ROWS_PER_CHUNK + h]],
                rows[p].at[pl.ds(h * IDX_MINOR, IDX_MINOR)],
                gsem[p],
            ))
        return gs

    def add_pos(ref):
        def body(j, _):
            for k in range(EMB // 16):
                sl = pl.ds(k * 16, 16)
                ref[j, sl] = ref[j, sl] + pos_v[j, sl]
            return 0
        lax.fori_loop(0, CHUNK, body, 0)

    pending_g = [None, None]
    pending_w = [None, None]

    for c in range(NCHUNK + 1):
        if c < NCHUNK:
            p = c % 2
            if pending_w[p] is not None:
                pending_w[p].wait()
                pending_w[p] = None
            pending_g[p] = start_gather(c)
        if c >= 1:
            q = (c - 1) % 2
            for g in pending_g[q]:
                g.wait()
            pending_g[q] = None
            add_pos(rows[q])
            pending_w[q] = pltpu.async_copy(
                rows[q],
                out_hbm.at[pl.ds(row0 + (c - 1) * CHUNK, CHUNK)],
                wsem[q],
            )

    for p in range(2):
        if pending_w[p] is not None:
            pending_w[p].wait()


def kernel(inputs, token_table, position_table):
    batch, seq = inputs.shape
    idx = inputs.reshape(-1).astype(jnp.int32).reshape(TOTAL // IDX_MINOR, IDX_MINOR)
    out = _embed_sc(idx, token_table, position_table)
    return out.reshape(batch, seq, EMB)
